# Initial kernel scaffold; baseline (speedup 1.0000x reference)
import jax
import jax.numpy as jnp
from jax.experimental import pallas as pl


def kernel(x, qkv_w, qkv_b, proj_w, proj_b, temperature):
    raise NotImplementedError("write your pallas kernel here")


# speedup vs baseline: 14.4089x; 14.4089x over previous
import functools

import jax
import jax.numpy as jnp
from jax.experimental import pallas as pl
from jax.experimental.pallas import tpu as pltpu

_DIM = 1024
_H = 16
_HD = _DIM // _H
_KEEP_FRAC = 0.5
_SCALE = _HD ** (-0.5)
_T_BISECT = 16


def _qkv_kernel(x_ref, w_ref, b_ref, t_ref, o_ref):
    j = pl.program_id(0)
    res = jax.lax.dot_general(
        x_ref[...], w_ref[...], (((1,), (1,)), ((), ())),
        preferred_element_type=jnp.float32)
    res = res + b_ref[...]
    inv = 1.0 / jnp.maximum(t_ref[0, 0], 0.01)
    o_ref[...] = jnp.where(j == 0, res * inv, res)


def _attn_kernel(q_ref, k_ref, v_ref, o_ref, *, keep):
    q = q_ref[0]
    k = k_ref[0]
    v = v_ref[0]
    s = jax.lax.dot_general(
        q, k, (((1,), (1,)), ((), ())),
        preferred_element_type=jnp.float32) * _SCALE
    lo = jnp.min(s, axis=1, keepdims=True)
    hi = jnp.max(s, axis=1, keepdims=True)
    m = hi

    def body(_, carry):
        lo, hi = carry
        mid = 0.5 * (lo + hi)
        cnt = jnp.sum(jnp.where(s >= mid, 1.0, 0.0), axis=1, keepdims=True)
        ge = cnt >= keep
        return jnp.where(ge, mid, lo), jnp.where(ge, hi, mid)

    lo, hi = jax.lax.fori_loop(0, _T_BISECT, body, (lo, hi))
    e = jnp.where(s >= lo, jnp.exp(s - m), 0.0)
    denom = jnp.sum(e, axis=1, keepdims=True)
    av = jax.lax.dot_general(
        e, v, (((1,), (0,)), ((), ())),
        preferred_element_type=jnp.float32)
    o_ref[0] = av / denom


def _proj_kernel(a_ref, w_ref, b_ref, o_ref):
    o_ref[...] = jax.lax.dot_general(
        a_ref[...], w_ref[...], (((1,), (1,)), ((), ())),
        preferred_element_type=jnp.float32) + b_ref[...]


def kernel(x, qkv_w, qkv_b, proj_w, proj_b, temperature):
    B, N, C = x.shape
    keep = max(1, int(N * _KEEP_FRAC))
    x2 = x.reshape(N, C).astype(jnp.float32)
    temp = temperature.reshape(1, 1).astype(jnp.float32)

    qkv = pl.pallas_call(
        _qkv_kernel,
        grid=(3,),
        in_specs=[
            pl.BlockSpec((N, C), lambda j: (0, 0)),
            pl.BlockSpec((C, C), lambda j: (j, 0)),
            pl.BlockSpec((1, C), lambda j: (0, j)),
            pl.BlockSpec((1, 1), lambda j: (0, 0)),
        ],
        out_specs=pl.BlockSpec((N, C), lambda j: (0, j)),
        out_shape=jax.ShapeDtypeStruct((N, 3 * C), jnp.float32),
    )(x2, qkv_w, qkv_b.reshape(1, 3 * C), temp)

    qkvh = qkv.reshape(N, 3, _H, _HD).transpose(1, 2, 0, 3)
    q3, k3, v3 = qkvh[0], qkvh[1], qkvh[2]

    attn = pl.pallas_call(
        functools.partial(_attn_kernel, keep=keep),
        grid=(_H,),
        in_specs=[pl.BlockSpec((1, N, _HD), lambda h: (h, 0, 0))] * 3,
        out_specs=pl.BlockSpec((1, N, _HD), lambda h: (h, 0, 0)),
        out_shape=jax.ShapeDtypeStruct((_H, N, _HD), jnp.float32),
    )(q3, k3, v3)

    attn2 = attn.transpose(1, 0, 2).reshape(N, C)

    out = pl.pallas_call(
        _proj_kernel,
        in_specs=[
            pl.BlockSpec((N, C), lambda: (0, 0)),
            pl.BlockSpec((C, C), lambda: (0, 0)),
            pl.BlockSpec((1, C), lambda: (0, 0)),
        ],
        out_specs=pl.BlockSpec((N, C), lambda: (0, 0)),
        out_shape=jax.ShapeDtypeStruct((N, C), jnp.float32),
    )(attn2, proj_w, proj_b.reshape(1, C))

    return out.reshape(B, N, C)


# speedup vs baseline: 18.0067x; 1.2497x over previous
import functools

import jax
import jax.numpy as jnp
from jax.experimental import pallas as pl
from jax.experimental.pallas import tpu as pltpu

_DIM = 1024
_H = 16
_HD = _DIM // _H
_KEEP_FRAC = 0.5
_SCALE = _HD ** (-0.5)
_T_BISECT = 10


def _qkv_kernel(x_ref, w_ref, b_ref, t_ref, o_ref):
    j = pl.program_id(0)
    res = jax.lax.dot_general(
        x_ref[...], w_ref[...], (((1,), (1,)), ((), ())),
        preferred_element_type=jnp.float32)
    res = res + b_ref[...]
    inv = _SCALE / jnp.maximum(t_ref[0, 0], 0.01)
    o_ref[...] = jnp.where(j == 0, res * inv, res)


def _attn_kernel(q_ref, k_ref, v_ref, o_ref, *, keep):
    q = q_ref[0]
    k = k_ref[0]
    v = v_ref[0]
    s = jax.lax.dot_general(
        q, k, (((1,), (1,)), ((), ())),
        preferred_element_type=jnp.float32)
    sb = s.astype(jnp.bfloat16)
    lo = jnp.min(s, axis=1, keepdims=True)
    hi = jnp.max(s, axis=1, keepdims=True)
    m = hi

    def body(_, carry):
        lo, hi = carry
        mid = 0.5 * (lo + hi)
        ind = jnp.where(sb >= mid.astype(jnp.bfloat16),
                        jnp.bfloat16(1), jnp.bfloat16(0))
        cnt = jnp.sum(ind, axis=1, keepdims=True, dtype=jnp.float32)
        ge = cnt >= keep
        return jnp.where(ge, mid, lo), jnp.where(ge, hi, mid)

    lo, hi = jax.lax.fori_loop(0, _T_BISECT, body, (lo, hi))
    e = jnp.where(s >= lo, jnp.exp(s - m), 0.0)
    denom = jnp.sum(e, axis=1, keepdims=True)
    av = jax.lax.dot_general(
        e, v, (((1,), (0,)), ((), ())),
        preferred_element_type=jnp.float32)
    o_ref[0] = av / denom


def _proj_kernel(a_ref, w_ref, b_ref, o_ref):
    o_ref[...] = jax.lax.dot_general(
        a_ref[...], w_ref[...], (((1,), (1,)), ((), ())),
        preferred_element_type=jnp.float32) + b_ref[...]


def kernel(x, qkv_w, qkv_b, proj_w, proj_b, temperature):
    B, N, C = x.shape
    keep = max(1, int(N * _KEEP_FRAC))
    x2 = x.reshape(N, C).astype(jnp.float32)
    temp = temperature.reshape(1, 1).astype(jnp.float32)

    qkv = pl.pallas_call(
        _qkv_kernel,
        grid=(3,),
        in_specs=[
            pl.BlockSpec((N, C), lambda j: (0, 0)),
            pl.BlockSpec((C, C), lambda j: (j, 0)),
            pl.BlockSpec((1, C), lambda j: (0, j)),
            pl.BlockSpec((1, 1), lambda j: (0, 0)),
        ],
        out_specs=pl.BlockSpec((N, C), lambda j: (0, j)),
        out_shape=jax.ShapeDtypeStruct((N, 3 * C), jnp.float32),
    )(x2, qkv_w, qkv_b.reshape(1, 3 * C), temp)

    qkvh = qkv.reshape(N, 3, _H, _HD).transpose(1, 2, 0, 3)
    q3, k3, v3 = qkvh[0], qkvh[1], qkvh[2]

    attn = pl.pallas_call(
        functools.partial(_attn_kernel, keep=keep),
        grid=(_H,),
        in_specs=[pl.BlockSpec((1, N, _HD), lambda h: (h, 0, 0))] * 3,
        out_specs=pl.BlockSpec((1, N, _HD), lambda h: (h, 0, 0)),
        out_shape=jax.ShapeDtypeStruct((_H, N, _HD), jnp.float32),
    )(q3, k3, v3)

    attn2 = attn.transpose(1, 0, 2).reshape(N, C)

    out = pl.pallas_call(
        _proj_kernel,
        in_specs=[
            pl.BlockSpec((N, C), lambda: (0, 0)),
            pl.BlockSpec((C, C), lambda: (0, 0)),
            pl.BlockSpec((1, C), lambda: (0, 0)),
        ],
        out_specs=pl.BlockSpec((N, C), lambda: (0, 0)),
        out_shape=jax.ShapeDtypeStruct((N, C), jnp.float32),
    )(attn2, proj_w, proj_b.reshape(1, C))

    return out.reshape(B, N, C)


# speedup vs baseline: 22.7940x; 1.2659x over previous
import functools

import jax
import jax.numpy as jnp
from jax.experimental import pallas as pl
from jax.experimental.pallas import tpu as pltpu

_DIM = 1024
_H = 16
_HD = _DIM // _H
_KEEP_FRAC = 0.5
_SCALE = _HD ** (-0.5)
_T_BISECT = 10


def _qkv_kernel(x_ref, w_ref, b_ref, t_ref, o_ref):
    j = pl.program_id(0)
    res = jax.lax.dot_general(
        x_ref[...], w_ref[...], (((1,), (1,)), ((), ())),
        preferred_element_type=jnp.float32)
    res = res + b_ref[...]
    inv = _SCALE / jnp.maximum(t_ref[0, 0], 0.01)
    o_ref[...] = jnp.where(j == 0, res * inv, res)


def _head_attention(q, k, v, keep):
    s = jax.lax.dot_general(
        q, k, (((1,), (1,)), ((), ())),
        preferred_element_type=jnp.float32)
    sb = s.astype(jnp.bfloat16)
    lo = jnp.min(s, axis=1, keepdims=True)
    hi = jnp.max(s, axis=1, keepdims=True)
    m = hi

    def body(_, carry):
        lo, hi = carry
        mid = 0.5 * (lo + hi)
        ind = jnp.where(sb >= mid.astype(jnp.bfloat16),
                        jnp.bfloat16(1), jnp.bfloat16(0))
        cnt = jnp.sum(ind, axis=1, keepdims=True, dtype=jnp.float32)
        ge = cnt >= keep
        return jnp.where(ge, mid, lo), jnp.where(ge, hi, mid)

    lo, hi = jax.lax.fori_loop(0, _T_BISECT, body, (lo, hi))
    e = jnp.where(s >= lo, jnp.exp(s - m), 0.0)
    denom = jnp.sum(e, axis=1, keepdims=True)
    av = jax.lax.dot_general(
        e.astype(jnp.bfloat16), v.astype(jnp.bfloat16),
        (((1,), (0,)), ((), ())),
        preferred_element_type=jnp.float32)
    return av / denom


def _attn_kernel(q_ref, k_ref, v_ref, o_ref, *, keep):
    qq = q_ref[...]
    kk = k_ref[...]
    vv = v_ref[...]
    outs = []
    for i in range(2):
        sl = slice(i * _HD, (i + 1) * _HD)
        outs.append(_head_attention(qq[:, sl], kk[:, sl], vv[:, sl], keep))
    o_ref[...] = jnp.concatenate(outs, axis=1)


def _proj_kernel(a_ref, w_ref, b_ref, o_ref):
    o_ref[...] = jax.lax.dot_general(
        a_ref[...], w_ref[...], (((1,), (1,)), ((), ())),
        preferred_element_type=jnp.float32) + b_ref[...]


def kernel(x, qkv_w, qkv_b, proj_w, proj_b, temperature):
    B, N, C = x.shape
    keep = max(1, int(N * _KEEP_FRAC))
    x2 = x.reshape(N, C).astype(jnp.float32)
    temp = temperature.reshape(1, 1).astype(jnp.float32)

    qkv = pl.pallas_call(
        _qkv_kernel,
        grid=(3,),
        in_specs=[
            pl.BlockSpec((N, C), lambda j: (0, 0)),
            pl.BlockSpec((C, C), lambda j: (j, 0)),
            pl.BlockSpec((1, C), lambda j: (0, j)),
            pl.BlockSpec((1, 1), lambda j: (0, 0)),
        ],
        out_specs=pl.BlockSpec((N, C), lambda j: (0, j)),
        out_shape=jax.ShapeDtypeStruct((N, 3 * C), jnp.float32),
    )(x2, qkv_w, qkv_b.reshape(1, 3 * C), temp)

    npairs = _H // 2
    attn = pl.pallas_call(
        functools.partial(_attn_kernel, keep=keep),
        grid=(npairs,),
        in_specs=[
            pl.BlockSpec((N, 2 * _HD), lambda g: (0, g)),
            pl.BlockSpec((N, 2 * _HD), lambda g: (0, npairs + g)),
            pl.BlockSpec((N, 2 * _HD), lambda g: (0, 2 * npairs + g)),
        ],
        out_specs=pl.BlockSpec((N, 2 * _HD), lambda g: (0, g)),
        out_shape=jax.ShapeDtypeStruct((N, C), jnp.float32),
    )(qkv, qkv, qkv)

    out = pl.pallas_call(
        _proj_kernel,
        in_specs=[
            pl.BlockSpec((N, C), lambda: (0, 0)),
            pl.BlockSpec((C, C), lambda: (0, 0)),
            pl.BlockSpec((1, C), lambda: (0, 0)),
        ],
        out_specs=pl.BlockSpec((N, C), lambda: (0, 0)),
        out_shape=jax.ShapeDtypeStruct((N, C), jnp.float32),
    )(attn, proj_w, proj_b.reshape(1, C))

    return out.reshape(B, N, C)
